# baseline (speedup 1.0000x reference)
import jax
import jax.numpy as jnp
from jax.experimental import pallas as pl


def _elu(x):
    return jnp.where(x > 0, x, jnp.exp(x) - 1.0)


def _fused_gcn_kernel(x_ref, w1_ref, w2_ref, o_ref):
    nb = x_ref.shape[0]
    xbs = [x_ref[b] for b in range(nb)]
    w1 = w1_ref[...]
    w2 = w2_ref[...]
    hws = [jnp.dot(xb, w1, preferred_element_type=jnp.float32).astype(jnp.bfloat16)
           for xb in xbs]
    as_ = []
    for xb in xbs:
        g = jnp.dot(xb, xb.T, preferred_element_type=jnp.float32)
        as_.append(jnp.tanh(jax.nn.relu(g)).astype(jnp.bfloat16))
    h1s = [_elu(jnp.dot(a, hw, preferred_element_type=jnp.float32))
           for a, hw in zip(as_, hws)]
    hw2s = [jnp.dot(h1, w2, preferred_element_type=jnp.float32).astype(jnp.bfloat16)
            for h1 in h1s]
    h2s = [_elu(jnp.dot(a, hw, preferred_element_type=jnp.float32))
           for a, hw in zip(as_, hw2s)]
    for b in range(nb):
        o_ref[b] = h2s[b]


def kernel(X, W1, W2):
    Bv, NTv, Dv = X.shape
    Xb = X.astype(jnp.bfloat16)
    Wb1 = W1.astype(jnp.bfloat16)
    out = pl.pallas_call(
        _fused_gcn_kernel,
        out_shape=jax.ShapeDtypeStruct((Bv, NTv, Dv), jnp.float32),
    )(Xb, Wb1, W2)
    return out


# speedup vs baseline: 1.0807x; 1.0807x over previous
import jax
import jax.numpy as jnp
from jax.experimental import pallas as pl


def _elu(x):
    return jnp.where(x > 0, x, jnp.exp(x) - 1.0)


def _fused_gcn_kernel(x_ref, w1_ref, w2_ref, o_ref):
    nb = x_ref.shape[0]
    xs = [x_ref[b] for b in range(nb)]
    w1 = w1_ref[...]
    w2 = w2_ref[...]
    hw1s = [jnp.dot(x, w1, preferred_element_type=jnp.float32).astype(jnp.bfloat16)
            for x in xs]
    as_ = []
    for x in xs:
        xb = x.astype(jnp.bfloat16)
        g = jnp.dot(xb, xb.T, preferred_element_type=jnp.float32)
        as_.append(jnp.tanh(jax.nn.relu(g)).astype(jnp.bfloat16))
    h1s = [_elu(jnp.dot(a, hw, preferred_element_type=jnp.float32))
           for a, hw in zip(as_, hw1s)]
    hw2s = [jnp.dot(h1, w2, preferred_element_type=jnp.float32).astype(jnp.bfloat16)
            for h1 in h1s]
    h2s = [_elu(jnp.dot(a, hw, preferred_element_type=jnp.float32))
           for a, hw in zip(as_, hw2s)]
    for b in range(nb):
        o_ref[b] = h2s[b]


def kernel(X, W1, W2):
    Bv, NTv, Dv = X.shape
    out = pl.pallas_call(
        _fused_gcn_kernel,
        out_shape=jax.ShapeDtypeStruct((Bv, NTv, Dv), jnp.float32),
    )(X, W1, W2)
    return out
